# baseline (speedup 1.0000x reference)
import dataclasses
import functools

import jax
import jax.numpy as jnp
from jax import lax
from jax.experimental import pallas as pl
from jax.experimental.pallas import tpu as pltpu
from jax.experimental.pallas import tpu_sc as plsc

B, S, H = 1, 2048, 1024
E, K, I = 8, 2, 512
T = B * S
TILE = 512
NTR = 16
NSH = T // TILE
NTILES = NTR + NSH
ROWS_R = NTR * TILE
ROWS = ROWS_R + T

NC, NS = 2, 16
NW = NC * NS


def _router_kernel(x_ref, wg_ref, pw_ref, pos_ref, te_ref):
    logits = jnp.dot(x_ref[...], wg_ref[...].T,
                     preferred_element_type=jnp.float32)
    m1 = jnp.max(logits, axis=-1, keepdims=True)
    sel1 = (logits == m1).astype(jnp.float32)
    masked = jnp.where(sel1 > 0, -jnp.inf, logits)
    m2 = jnp.max(masked, axis=-1, keepdims=True)
    sel2 = (masked == m2).astype(jnp.float32)

    e2 = jnp.exp(m2 - m1)
    denom = 1.0 + e2 + 1e-20
    w1 = 1.0 / denom
    w2 = e2 / denom
    col = lax.broadcasted_iota(jnp.int32, (T, E), 1)
    pw_ref[...] = jnp.where(col == 0, w1, jnp.where(col == 1, w2, 0.0))

    def _cumsum0(a):
        c = a
        k = 1
        while k < T:
            c = c + jnp.concatenate(
                [jnp.zeros((k, E), a.dtype), c[:-k]], axis=0)
            k *= 2
        return c

    c1 = _cumsum0(sel1)
    c2 = _cumsum0(sel2)
    rank1 = c1 - sel1
    rank2 = c2 - sel2
    count1 = c1[-1:, :]
    count2 = c2[-1:, :]
    count = count1 + count2
    tiles = jnp.floor((count + (TILE - 1)) * (1.0 / TILE))
    lo = lax.broadcasted_iota(jnp.int32, (E, E), 0)
    hi = lax.broadcasted_iota(jnp.int32, (E, E), 1)
    cumt = jnp.dot(tiles, (lo <= hi).astype(jnp.float32),
                   preferred_element_type=jnp.float32)
    row_start = TILE * (cumt - tiles)

    p1 = jnp.sum(sel1 * (row_start + rank1), axis=1, keepdims=True)
    p2 = jnp.sum(sel2 * (row_start + count1 + rank2), axis=1, keepdims=True)
    pos_ref[...] = jnp.where(col == 0, p1.astype(jnp.int32),
                             jnp.where(col == 1, p2.astype(jnp.int32), 0))

    ti = lax.broadcasted_iota(jnp.int32, (NTR, E), 0).astype(jnp.float32)
    te = jnp.sum((ti >= cumt).astype(jnp.int32), axis=1, keepdims=True)
    te_ref[...] = jnp.broadcast_to(te, (NTR, E))


def _scatter_kernel(pos_hbm, src_hbm, pos_v, src_v):
    wid = lax.axis_index("s") * NC + lax.axis_index("c")

    @pl.when(wid == 0)
    def _():
        pltpu.sync_copy(pos_hbm, pos_v)
        iota16 = lax.iota(jnp.int32, 16)
        zeros16 = jnp.zeros((16,), jnp.int32)

        @pl.loop(0, ROWS_R, step=16)
        def _(i):
            src_v[pl.ds(i, 16)] = zeros16

        @pl.loop(0, T, step=16)
        def _(i):
            src_v[pl.ds(ROWS_R + i, 16)] = iota16 + i

        @pl.loop(0, T, step=16)
        def _(i):
            tok = iota16 + i
            p1 = plsc.load_gather(pos_v, [tok * E])
            plsc.store_scatter(src_v, [p1], tok)
            p2 = plsc.load_gather(pos_v, [tok * E + 1])
            plsc.store_scatter(src_v, [p2], tok)

        pltpu.sync_copy(src_v, src_hbm)


def _gather_kernel(n_rows, chunk, table_hbm, idx_hbm, out_hbm,
                   idx_v, rows_v, sem):
    wid = lax.axis_index("s") * NC + lax.axis_index("c")
    per_w = n_rows // NW
    base = wid * per_w

    @pl.loop(0, per_w, step=chunk)
    def _(c):
        b = base + c
        pltpu.sync_copy(idx_hbm.at[pl.ds(b, chunk)], idx_v)
        pltpu.async_copy(table_hbm.at[idx_v], rows_v, sem).wait()
        pltpu.sync_copy(rows_v, out_hbm.at[pl.ds(b, chunk)])


def _ffn_kernel(pf_ref, x_ref, wg_ref, wu_ref, wd_ref, o_ref):
    i = pl.program_id(0)

    @pl.when(pf_ref[i] <= E)
    def _():
        xb = x_ref[...].astype(jnp.bfloat16)
        g = jnp.dot(xb, wg_ref[0].T, preferred_element_type=jnp.float32)
        u = jnp.dot(xb, wu_ref[0].T, preferred_element_type=jnp.float32)
        hmid = (g * jax.nn.sigmoid(g)) * u
        o_ref[...] = jnp.dot(hmid.astype(jnp.bfloat16), wd_ref[0].T,
                             preferred_element_type=jnp.float32)


def _combine_kernel(pw_ref, r1_ref, r2_ref, sh_ref, o_ref):
    col = lax.broadcasted_iota(jnp.int32, (TILE, E), 1)
    pw = pw_ref[...]
    w1 = jnp.sum(jnp.where(col == 0, pw, 0.0), axis=1, keepdims=True)
    w2 = jnp.sum(jnp.where(col == 1, pw, 0.0), axis=1, keepdims=True)
    o_ref[...] = w1 * r1_ref[...] + w2 * r2_ref[...] + sh_ref[...]


@jax.jit
def kernel(x, Wg, We_gate, We_up, We_down, Ws_gate, Ws_up, Ws_down):
    xf = x.reshape(T, H)

    pw, pos, te_mat = pl.pallas_call(
        _router_kernel,
        out_shape=(
            jax.ShapeDtypeStruct((T, E), jnp.float32),
            jax.ShapeDtypeStruct((T, E), jnp.int32),
            jax.ShapeDtypeStruct((NTR, E), jnp.int32),
        ),
    )(xf, Wg)

    mesh = plsc.VectorSubcoreMesh(core_axis_name="c", subcore_axis_name="s")
    sc_params = pltpu.CompilerParams()
    if "needs_layout_passes" in pltpu.CompilerParams.__dataclass_fields__:
        sc_params = dataclasses.replace(sc_params, needs_layout_passes=False)

    src = pl.kernel(
        _scatter_kernel,
        out_type=jax.ShapeDtypeStruct((ROWS,), jnp.int32),
        mesh=mesh,
        scratch_types=[pltpu.VMEM((T * E,), jnp.int32),
                       pltpu.VMEM((ROWS,), jnp.int32)],
        compiler_params=sc_params,
    )(pos.reshape(-1))

    x_sorted = pl.kernel(
        functools.partial(_gather_kernel, ROWS, 64),
        out_type=jax.ShapeDtypeStruct((ROWS, H), jnp.float32),
        mesh=mesh,
        scratch_types=[pltpu.VMEM((64,), jnp.int32),
                       pltpu.VMEM((64, H), jnp.float32),
                       pltpu.SemaphoreType.DMA],
        compiler_params=sc_params,
    )(xf, src)

    te = te_mat[:, 0]
    pf = jnp.concatenate([jnp.where(te >= E, E + 1, te),
                          jnp.full((NSH,), E, jnp.int32)]).astype(jnp.int32)

    wcat_g = jnp.concatenate([We_gate, Ws_gate[None]], 0).astype(jnp.bfloat16)
    wcat_u = jnp.concatenate([We_up, Ws_up[None]], 0).astype(jnp.bfloat16)
    wcat_d = jnp.concatenate([We_down, Ws_down[None]], 0).astype(jnp.bfloat16)

    os_ = pl.pallas_call(
        _ffn_kernel,
        grid_spec=pltpu.PrefetchScalarGridSpec(
            num_scalar_prefetch=1,
            grid=(NTILES,),
            in_specs=[
                pl.BlockSpec((TILE, H), lambda i, pf: (i, 0)),
                pl.BlockSpec((1, I, H),
                             lambda i, pf: (jnp.minimum(pf[i], E), 0, 0)),
                pl.BlockSpec((1, I, H),
                             lambda i, pf: (jnp.minimum(pf[i], E), 0, 0)),
                pl.BlockSpec((1, H, I),
                             lambda i, pf: (jnp.minimum(pf[i], E), 0, 0)),
            ],
            out_specs=pl.BlockSpec((TILE, H), lambda i, pf: (i, 0)),
        ),
        out_shape=jax.ShapeDtypeStruct((ROWS, H), jnp.float32),
        compiler_params=pltpu.CompilerParams(
            dimension_semantics=("arbitrary",),
        ),
    )(pf, x_sorted, wcat_g, wcat_u, wcat_d)

    pos1 = pos[:, 0]
    pos2 = pos[:, 1]

    def _pair_gather(os_hbm, i1_hbm, i2_hbm, r1_hbm, r2_hbm,
                     idx_v, rows_v, sem):
        wid = lax.axis_index("s") * NC + lax.axis_index("c")
        base = wid * (T // NW)
        pltpu.sync_copy(i1_hbm.at[pl.ds(base, T // NW)], idx_v)
        pltpu.async_copy(os_hbm.at[idx_v], rows_v, sem).wait()
        pltpu.sync_copy(rows_v, r1_hbm.at[pl.ds(base, T // NW)])
        pltpu.sync_copy(i2_hbm.at[pl.ds(base, T // NW)], idx_v)
        pltpu.async_copy(os_hbm.at[idx_v], rows_v, sem).wait()
        pltpu.sync_copy(rows_v, r2_hbm.at[pl.ds(base, T // NW)])

    r1, r2 = pl.kernel(
        _pair_gather,
        out_type=(jax.ShapeDtypeStruct((T, H), jnp.float32),
                  jax.ShapeDtypeStruct((T, H), jnp.float32)),
        mesh=mesh,
        scratch_types=[pltpu.VMEM((T // NW,), jnp.int32),
                       pltpu.VMEM((T // NW, H), jnp.float32),
                       pltpu.SemaphoreType.DMA],
        compiler_params=sc_params,
    )(os_, pos1, pos2)

    y = pl.pallas_call(
        _combine_kernel,
        grid=(T // TILE,),
        in_specs=[
            pl.BlockSpec((TILE, E), lambda i: (i, 0)),
            pl.BlockSpec((TILE, H), lambda i: (i, 0)),
            pl.BlockSpec((TILE, H), lambda i: (i, 0)),
            pl.BlockSpec((TILE, H), lambda i: (NTR + i, 0)),
        ],
        out_specs=pl.BlockSpec((TILE, H), lambda i: (i, 0)),
        out_shape=jax.ShapeDtypeStruct((T, H), jnp.float32),
    )(pw, r1, r2, os_)
    return y.reshape(B, S, H)


# speedup vs baseline: 1.0931x; 1.0931x over previous
import dataclasses
import functools

import jax
import jax.numpy as jnp
from jax import lax
from jax.experimental import pallas as pl
from jax.experimental.pallas import tpu as pltpu
from jax.experimental.pallas import tpu_sc as plsc

B, S, H = 1, 2048, 1024
E, K, I = 8, 2, 512
T = B * S
TILE = 512
NTR = 16
ROWS = NTR * TILE

NC, NS = 2, 16
NW = NC * NS


def _router_kernel(x_ref, wg_ref, pw_ref, pos_ref, te_ref):
    logits = jnp.dot(x_ref[...], wg_ref[...].T,
                     preferred_element_type=jnp.float32)
    m1 = jnp.max(logits, axis=-1, keepdims=True)
    sel1 = (logits == m1).astype(jnp.float32)
    masked = jnp.where(sel1 > 0, -jnp.inf, logits)
    m2 = jnp.max(masked, axis=-1, keepdims=True)
    sel2 = (masked == m2).astype(jnp.float32)

    e2 = jnp.exp(m2 - m1)
    denom = 1.0 + e2 + 1e-20
    w1 = 1.0 / denom
    w2 = e2 / denom
    col = lax.broadcasted_iota(jnp.int32, (T, E), 1)
    pw_ref[...] = jnp.where(col == 0, w1, jnp.where(col == 1, w2, 0.0))

    def _cumsum0(a):
        c = a
        k = 1
        while k < T:
            c = c + jnp.concatenate(
                [jnp.zeros((k, E), a.dtype), c[:-k]], axis=0)
            k *= 2
        return c

    c1 = _cumsum0(sel1)
    c2 = _cumsum0(sel2)
    rank1 = c1 - sel1
    rank2 = c2 - sel2
    count1 = c1[-1:, :]
    count2 = c2[-1:, :]
    count = count1 + count2
    tiles = jnp.floor((count + (TILE - 1)) * (1.0 / TILE))
    lo = lax.broadcasted_iota(jnp.int32, (E, E), 0)
    hi = lax.broadcasted_iota(jnp.int32, (E, E), 1)
    cumt = jnp.dot(tiles, (lo <= hi).astype(jnp.float32),
                   preferred_element_type=jnp.float32)
    row_start = TILE * (cumt - tiles)

    p1 = jnp.sum(sel1 * (row_start + rank1), axis=1, keepdims=True)
    p2 = jnp.sum(sel2 * (row_start + count1 + rank2), axis=1, keepdims=True)
    pos_ref[...] = jnp.where(col == 0, p1.astype(jnp.int32),
                             jnp.where(col == 1, p2.astype(jnp.int32), 0))

    ti = lax.broadcasted_iota(jnp.int32, (NTR, E), 0).astype(jnp.float32)
    te = jnp.sum((ti >= cumt).astype(jnp.int32), axis=1, keepdims=True)
    te_ref[...] = jnp.broadcast_to(te, (NTR, E))


def _scatter_kernel(pos_hbm, src_hbm, pos_v, src_v):
    wid = lax.axis_index("s") * NC + lax.axis_index("c")

    @pl.when(wid == 0)
    def _():
        pltpu.sync_copy(pos_hbm, pos_v)
        iota16 = lax.iota(jnp.int32, 16)
        zeros16 = jnp.zeros((16,), jnp.int32)

        @pl.loop(0, ROWS, step=16)
        def _(i):
            src_v[pl.ds(i, 16)] = zeros16

        @pl.loop(0, T, step=16)
        def _(i):
            tok = iota16 + i
            p1 = plsc.load_gather(pos_v, [tok * E])
            plsc.store_scatter(src_v, [p1], tok)
            p2 = plsc.load_gather(pos_v, [tok * E + 1])
            plsc.store_scatter(src_v, [p2], tok)

        pltpu.sync_copy(src_v, src_hbm)


def _gather_kernel(table_hbm, idx_hbm, out_hbm, idx_v, rows_v, sem):
    wid = lax.axis_index("s") * NC + lax.axis_index("c")
    per_w = ROWS // NW
    chunk = 64
    base = wid * per_w
    pltpu.sync_copy(idx_hbm.at[pl.ds(base, per_w)], idx_v)
    for k in range(per_w // chunk):
        pltpu.async_copy(table_hbm.at[idx_v.at[pl.ds(k * chunk, chunk)]],
                         rows_v, sem).wait()
        pltpu.sync_copy(rows_v, out_hbm.at[pl.ds(base + k * chunk, chunk)])


def _ffn_body(x_ref, wg_ref, wu_ref, wd_ref, o_ref):
    xb = x_ref[...].astype(jnp.bfloat16)
    g = jnp.dot(xb, wg_ref[0].T, preferred_element_type=jnp.float32)
    u = jnp.dot(xb, wu_ref[0].T, preferred_element_type=jnp.float32)
    hmid = (g * jax.nn.sigmoid(g)) * u
    o_ref[...] = jnp.dot(hmid.astype(jnp.bfloat16), wd_ref[0].T,
                         preferred_element_type=jnp.float32)


def _ffn_routed_kernel(pf_ref, x_ref, wg_ref, wu_ref, wd_ref, o_ref):
    @pl.when(pf_ref[pl.program_id(0)] < E)
    def _():
        _ffn_body(x_ref, wg_ref, wu_ref, wd_ref, o_ref)


def _combine_kernel(pw_ref, r1_ref, r2_ref, sh_ref, o_ref):
    col = lax.broadcasted_iota(jnp.int32, (TILE, E), 1)
    pw = pw_ref[...]
    w1 = jnp.sum(jnp.where(col == 0, pw, 0.0), axis=1, keepdims=True)
    w2 = jnp.sum(jnp.where(col == 1, pw, 0.0), axis=1, keepdims=True)
    o_ref[...] = w1 * r1_ref[...] + w2 * r2_ref[...] + sh_ref[...]


@jax.jit
def kernel(x, Wg, We_gate, We_up, We_down, Ws_gate, Ws_up, Ws_down):
    xf = x.reshape(T, H)

    pw, pos, te_mat = pl.pallas_call(
        _router_kernel,
        out_shape=(
            jax.ShapeDtypeStruct((T, E), jnp.float32),
            jax.ShapeDtypeStruct((T, E), jnp.int32),
            jax.ShapeDtypeStruct((NTR, E), jnp.int32),
        ),
    )(xf, Wg)

    mesh = plsc.VectorSubcoreMesh(core_axis_name="c", subcore_axis_name="s")
    sc_params = pltpu.CompilerParams()
    if "needs_layout_passes" in pltpu.CompilerParams.__dataclass_fields__:
        sc_params = dataclasses.replace(sc_params, needs_layout_passes=False)

    src = pl.kernel(
        _scatter_kernel,
        out_type=jax.ShapeDtypeStruct((ROWS,), jnp.int32),
        mesh=mesh,
        scratch_types=[pltpu.VMEM((T * E,), jnp.int32),
                       pltpu.VMEM((ROWS,), jnp.int32)],
        compiler_params=sc_params,
    )(pos.reshape(-1))

    x_sorted = pl.kernel(
        _gather_kernel,
        out_type=jax.ShapeDtypeStruct((ROWS, H), jnp.float32),
        mesh=mesh,
        scratch_types=[pltpu.VMEM((ROWS // NW,), jnp.int32),
                       pltpu.VMEM((64, H), jnp.float32),
                       pltpu.SemaphoreType.DMA],
        compiler_params=sc_params,
    )(xf, src)

    te = te_mat[:, 0]
    pf = jnp.minimum(te, E).astype(jnp.int32)

    we_g = We_gate.astype(jnp.bfloat16)
    we_u = We_up.astype(jnp.bfloat16)
    we_d = We_down.astype(jnp.bfloat16)

    sh = pl.pallas_call(
        _ffn_body,
        grid=(T // TILE,),
        in_specs=[
            pl.BlockSpec((TILE, H), lambda i: (i, 0)),
            pl.BlockSpec((1, I, H), lambda i: (0, 0, 0)),
            pl.BlockSpec((1, I, H), lambda i: (0, 0, 0)),
            pl.BlockSpec((1, H, I), lambda i: (0, 0, 0)),
        ],
        out_specs=pl.BlockSpec((TILE, H), lambda i: (i, 0)),
        out_shape=jax.ShapeDtypeStruct((T, H), jnp.float32),
    )(xf, Ws_gate.astype(jnp.bfloat16)[None],
      Ws_up.astype(jnp.bfloat16)[None], Ws_down.astype(jnp.bfloat16)[None])

    os_ = pl.pallas_call(
        _ffn_routed_kernel,
        grid_spec=pltpu.PrefetchScalarGridSpec(
            num_scalar_prefetch=1,
            grid=(NTR,),
            in_specs=[
                pl.BlockSpec((TILE, H), lambda i, pf: (i, 0)),
                pl.BlockSpec((1, I, H),
                             lambda i, pf: (jnp.minimum(pf[i], E - 1), 0, 0)),
                pl.BlockSpec((1, I, H),
                             lambda i, pf: (jnp.minimum(pf[i], E - 1), 0, 0)),
                pl.BlockSpec((1, H, I),
                             lambda i, pf: (jnp.minimum(pf[i], E - 1), 0, 0)),
            ],
            out_specs=pl.BlockSpec((TILE, H), lambda i, pf: (i, 0)),
        ),
        out_shape=jax.ShapeDtypeStruct((ROWS, H), jnp.float32),
        compiler_params=pltpu.CompilerParams(
            dimension_semantics=("arbitrary",),
        ),
    )(pf, x_sorted, we_g, we_u, we_d)

    pos1 = pos[:, 0]
    pos2 = pos[:, 1]

    def _pair_gather(os_hbm, i1_hbm, i2_hbm, r1_hbm, r2_hbm,
                     idx_v, rows_v, sem):
        wid = lax.axis_index("s") * NC + lax.axis_index("c")
        base = wid * (T // NW)
        pltpu.sync_copy(i1_hbm.at[pl.ds(base, T // NW)], idx_v)
        pltpu.async_copy(os_hbm.at[idx_v], rows_v, sem).wait()
        pltpu.sync_copy(rows_v, r1_hbm.at[pl.ds(base, T // NW)])
        pltpu.sync_copy(i2_hbm.at[pl.ds(base, T // NW)], idx_v)
        pltpu.async_copy(os_hbm.at[idx_v], rows_v, sem).wait()
        pltpu.sync_copy(rows_v, r2_hbm.at[pl.ds(base, T // NW)])

    r1, r2 = pl.kernel(
        _pair_gather,
        out_type=(jax.ShapeDtypeStruct((T, H), jnp.float32),
                  jax.ShapeDtypeStruct((T, H), jnp.float32)),
        mesh=mesh,
        scratch_types=[pltpu.VMEM((T // NW,), jnp.int32),
                       pltpu.VMEM((T // NW, H), jnp.float32),
                       pltpu.SemaphoreType.DMA],
        compiler_params=sc_params,
    )(os_, pos1, pos2)

    y = pl.pallas_call(
        _combine_kernel,
        grid=(T // TILE,),
        in_specs=[
            pl.BlockSpec((TILE, E), lambda i: (i, 0)),
            pl.BlockSpec((TILE, H), lambda i: (i, 0)),
            pl.BlockSpec((TILE, H), lambda i: (i, 0)),
            pl.BlockSpec((TILE, H), lambda i: (i, 0)),
        ],
        out_specs=pl.BlockSpec((TILE, H), lambda i: (i, 0)),
        out_shape=jax.ShapeDtypeStruct((T, H), jnp.float32),
    )(pw, r1, r2, sh)
    return y.reshape(B, S, H)


# speedup vs baseline: 2.4356x; 2.2281x over previous
import dataclasses
import functools

import jax
import jax.numpy as jnp
from jax import lax
from jax.experimental import pallas as pl
from jax.experimental.pallas import tpu as pltpu
from jax.experimental.pallas import tpu_sc as plsc

B, S, H = 1, 2048, 1024
E, K, I = 8, 2, 512
T = B * S
TILE = 512
NTR = 16
ROWS = NTR * TILE

NC, NS = 2, 16
NW = NC * NS


def _router_kernel(x_ref, wg_ref, pw_ref, pos_ref, te_ref):
    logits = jnp.dot(x_ref[...], wg_ref[...].T,
                     preferred_element_type=jnp.float32)
    m1 = jnp.max(logits, axis=-1, keepdims=True)
    sel1 = (logits == m1).astype(jnp.float32)
    masked = jnp.where(sel1 > 0, -jnp.inf, logits)
    m2 = jnp.max(masked, axis=-1, keepdims=True)
    sel2 = (masked == m2).astype(jnp.float32)

    e2 = jnp.exp(m2 - m1)
    denom = 1.0 + e2 + 1e-20
    w1 = 1.0 / denom
    w2 = e2 / denom
    col = lax.broadcasted_iota(jnp.int32, (T, E), 1)
    pw_ref[...] = jnp.where(col == 0, w1, jnp.where(col == 1, w2, 0.0))

    def _cumsum0(a):
        c = a
        k = 1
        while k < T:
            c = c + jnp.concatenate(
                [jnp.zeros((k, E), a.dtype), c[:-k]], axis=0)
            k *= 2
        return c

    c1 = _cumsum0(sel1)
    c2 = _cumsum0(sel2)
    rank1 = c1 - sel1
    rank2 = c2 - sel2
    count1 = c1[-1:, :]
    count2 = c2[-1:, :]
    count = count1 + count2
    tiles = jnp.floor((count + (TILE - 1)) * (1.0 / TILE))
    lo = lax.broadcasted_iota(jnp.int32, (E, E), 0)
    hi = lax.broadcasted_iota(jnp.int32, (E, E), 1)
    cumt = jnp.dot(tiles, (lo <= hi).astype(jnp.float32),
                   preferred_element_type=jnp.float32)
    row_start = TILE * (cumt - tiles)

    p1 = jnp.sum(sel1 * (row_start + rank1), axis=1, keepdims=True)
    p2 = jnp.sum(sel2 * (row_start + count1 + rank2), axis=1, keepdims=True)
    pos_ref[...] = jnp.where(col == 0, p1.astype(jnp.int32),
                             jnp.where(col == 1, p2.astype(jnp.int32), 0))

    ti = lax.broadcasted_iota(jnp.int32, (NTR, E), 0).astype(jnp.float32)
    te = jnp.sum((ti >= cumt).astype(jnp.int32), axis=1, keepdims=True)
    te_ref[...] = jnp.broadcast_to(te, (NTR, E))


def _scatter_kernel(pos_hbm, src_hbm, pos_v, src_v):
    wid = lax.axis_index("s") * NC + lax.axis_index("c")

    @pl.when(wid == 0)
    def _():
        pltpu.sync_copy(pos_hbm, pos_v)
        iota16 = lax.iota(jnp.int32, 16)

        @pl.loop(0, ROWS, step=16)
        def _(i):
            src_v[pl.ds(i, 16)] = (iota16 + i) & (T - 1)

        @pl.loop(0, T, step=16)
        def _(i):
            tok = iota16 + i
            p1 = plsc.load_gather(pos_v, [tok * E])
            plsc.store_scatter(src_v, [p1], tok)
            p2 = plsc.load_gather(pos_v, [tok * E + 1])
            plsc.store_scatter(src_v, [p2], tok)

        pltpu.sync_copy(src_v, src_hbm)


def _gather_kernel(table_hbm, idx_hbm, out_hbm, idx_v, rows_v, sem):
    wid = lax.axis_index("s") * NC + lax.axis_index("c")
    per_w = ROWS // NW
    chunk = 64
    base = wid * per_w
    pltpu.sync_copy(idx_hbm.at[pl.ds(base, per_w)], idx_v)
    for k in range(per_w // chunk):
        pltpu.async_copy(table_hbm.at[idx_v.at[pl.ds(k * chunk, chunk)]],
                         rows_v, sem).wait()
        pltpu.sync_copy(rows_v, out_hbm.at[pl.ds(base + k * chunk, chunk)])


def _ffn_body(x_ref, wg_ref, wu_ref, wd_ref, o_ref):
    xb = x_ref[...].astype(jnp.bfloat16)
    g = jnp.dot(xb, wg_ref[0].T, preferred_element_type=jnp.float32)
    u = jnp.dot(xb, wu_ref[0].T, preferred_element_type=jnp.float32)
    hmid = (g * jax.nn.sigmoid(g)) * u
    o_ref[...] = jnp.dot(hmid.astype(jnp.bfloat16), wd_ref[0].T,
                         preferred_element_type=jnp.float32)


def _ffn_routed_kernel(pf_ref, x_ref, wg_ref, wu_ref, wd_ref, o_ref):
    @pl.when(pf_ref[pl.program_id(0)] < E)
    def _():
        _ffn_body(x_ref, wg_ref, wu_ref, wd_ref, o_ref)


def _combine_kernel(pw_ref, r1_ref, r2_ref, sh_ref, o_ref):
    col = lax.broadcasted_iota(jnp.int32, (TILE, E), 1)
    pw = pw_ref[...]
    w1 = jnp.sum(jnp.where(col == 0, pw, 0.0), axis=1, keepdims=True)
    w2 = jnp.sum(jnp.where(col == 1, pw, 0.0), axis=1, keepdims=True)
    o_ref[...] = w1 * r1_ref[...] + w2 * r2_ref[...] + sh_ref[...]


@jax.jit
def kernel(x, Wg, We_gate, We_up, We_down, Ws_gate, Ws_up, Ws_down):
    xf = x.reshape(T, H)

    pw, pos, te_mat = pl.pallas_call(
        _router_kernel,
        out_shape=(
            jax.ShapeDtypeStruct((T, E), jnp.float32),
            jax.ShapeDtypeStruct((T, E), jnp.int32),
            jax.ShapeDtypeStruct((NTR, E), jnp.int32),
        ),
    )(xf, Wg)

    mesh = plsc.VectorSubcoreMesh(core_axis_name="c", subcore_axis_name="s")
    sc_params = pltpu.CompilerParams()
    if "needs_layout_passes" in pltpu.CompilerParams.__dataclass_fields__:
        sc_params = dataclasses.replace(sc_params, needs_layout_passes=False)

    src = pl.kernel(
        _scatter_kernel,
        out_type=jax.ShapeDtypeStruct((ROWS,), jnp.int32),
        mesh=mesh,
        scratch_types=[pltpu.VMEM((T * E,), jnp.int32),
                       pltpu.VMEM((ROWS,), jnp.int32)],
        compiler_params=sc_params,
    )(pos.reshape(-1))

    x_sorted = pl.kernel(
        _gather_kernel,
        out_type=jax.ShapeDtypeStruct((ROWS, H), jnp.float32),
        mesh=mesh,
        scratch_types=[pltpu.VMEM((ROWS // NW,), jnp.int32),
                       pltpu.VMEM((64, H), jnp.float32),
                       pltpu.SemaphoreType.DMA],
        compiler_params=sc_params,
    )(xf, src)

    te = te_mat[:, 0]
    pf = jnp.minimum(te, E).astype(jnp.int32)

    we_g = We_gate.astype(jnp.bfloat16)
    we_u = We_up.astype(jnp.bfloat16)
    we_d = We_down.astype(jnp.bfloat16)

    sh = pl.pallas_call(
        _ffn_body,
        grid=(T // TILE,),
        in_specs=[
            pl.BlockSpec((TILE, H), lambda i: (i, 0)),
            pl.BlockSpec((1, I, H), lambda i: (0, 0, 0)),
            pl.BlockSpec((1, I, H), lambda i: (0, 0, 0)),
            pl.BlockSpec((1, H, I), lambda i: (0, 0, 0)),
        ],
        out_specs=pl.BlockSpec((TILE, H), lambda i: (i, 0)),
        out_shape=jax.ShapeDtypeStruct((T, H), jnp.float32),
    )(xf, Ws_gate.astype(jnp.bfloat16)[None],
      Ws_up.astype(jnp.bfloat16)[None], Ws_down.astype(jnp.bfloat16)[None])

    os_ = pl.pallas_call(
        _ffn_routed_kernel,
        grid_spec=pltpu.PrefetchScalarGridSpec(
            num_scalar_prefetch=1,
            grid=(NTR,),
            in_specs=[
                pl.BlockSpec((TILE, H), lambda i, pf: (i, 0)),
                pl.BlockSpec((1, I, H),
                             lambda i, pf: (jnp.minimum(pf[i], E - 1), 0, 0)),
                pl.BlockSpec((1, I, H),
                             lambda i, pf: (jnp.minimum(pf[i], E - 1), 0, 0)),
                pl.BlockSpec((1, H, I),
                             lambda i, pf: (jnp.minimum(pf[i], E - 1), 0, 0)),
            ],
            out_specs=pl.BlockSpec((TILE, H), lambda i, pf: (i, 0)),
        ),
        out_shape=jax.ShapeDtypeStruct((ROWS, H), jnp.float32),
        compiler_params=pltpu.CompilerParams(
            dimension_semantics=("arbitrary",),
        ),
    )(pf, x_sorted, we_g, we_u, we_d)

    pos1 = pos[:, 0]
    pos2 = pos[:, 1]

    def _pair_gather(os_hbm, i1_hbm, i2_hbm, r1_hbm, r2_hbm,
                     idx_v, rows_v, sem):
        wid = lax.axis_index("s") * NC + lax.axis_index("c")
        base = wid * (T // NW)
        pltpu.sync_copy(i1_hbm.at[pl.ds(base, T // NW)], idx_v)
        pltpu.async_copy(os_hbm.at[idx_v], rows_v, sem).wait()
        pltpu.sync_copy(rows_v, r1_hbm.at[pl.ds(base, T // NW)])
        pltpu.sync_copy(i2_hbm.at[pl.ds(base, T // NW)], idx_v)
        pltpu.async_copy(os_hbm.at[idx_v], rows_v, sem).wait()
        pltpu.sync_copy(rows_v, r2_hbm.at[pl.ds(base, T // NW)])

    r1, r2 = pl.kernel(
        _pair_gather,
        out_type=(jax.ShapeDtypeStruct((T, H), jnp.float32),
                  jax.ShapeDtypeStruct((T, H), jnp.float32)),
        mesh=mesh,
        scratch_types=[pltpu.VMEM((T // NW,), jnp.int32),
                       pltpu.VMEM((T // NW, H), jnp.float32),
                       pltpu.SemaphoreType.DMA],
        compiler_params=sc_params,
    )(os_, pos1, pos2)

    y = pl.pallas_call(
        _combine_kernel,
        grid=(T // TILE,),
        in_specs=[
            pl.BlockSpec((TILE, E), lambda i: (i, 0)),
            pl.BlockSpec((TILE, H), lambda i: (i, 0)),
            pl.BlockSpec((TILE, H), lambda i: (i, 0)),
            pl.BlockSpec((TILE, H), lambda i: (i, 0)),
        ],
        out_specs=pl.BlockSpec((TILE, H), lambda i: (i, 0)),
        out_shape=jax.ShapeDtypeStruct((T, H), jnp.float32),
    )(pw, r1, r2, sh)
    return y.reshape(B, S, H)


# speedup vs baseline: 2.4677x; 1.0132x over previous
import dataclasses
import functools

import jax
import jax.numpy as jnp
from jax import lax
from jax.experimental import pallas as pl
from jax.experimental.pallas import tpu as pltpu
from jax.experimental.pallas import tpu_sc as plsc

B, S, H = 1, 2048, 1024
E, K, I = 8, 2, 512
T = B * S
TILE = 256
NTR = 24
ROWS = NTR * TILE
SH_TILE = 512

NC, NS = 2, 16
NW = NC * NS


def _router_kernel(x_ref, wg_ref, pw_ref, pos_ref, te_ref):
    logits = jnp.dot(x_ref[...], wg_ref[...].T,
                     preferred_element_type=jnp.float32)
    m1 = jnp.max(logits, axis=-1, keepdims=True)
    sel1 = (logits == m1).astype(jnp.float32)
    masked = jnp.where(sel1 > 0, -jnp.inf, logits)
    m2 = jnp.max(masked, axis=-1, keepdims=True)
    sel2 = (masked == m2).astype(jnp.float32)

    e2 = jnp.exp(m2 - m1)
    denom = 1.0 + e2 + 1e-20
    w1 = 1.0 / denom
    w2 = e2 / denom
    col = lax.broadcasted_iota(jnp.int32, (T, E), 1)
    pw_ref[...] = jnp.where(col == 0, w1, jnp.where(col == 1, w2, 0.0))

    def _cumsum0(a):
        c = a
        k = 1
        while k < T:
            c = c + jnp.concatenate(
                [jnp.zeros((k, E), a.dtype), c[:-k]], axis=0)
            k *= 2
        return c

    c1 = _cumsum0(sel1)
    c2 = _cumsum0(sel2)
    rank1 = c1 - sel1
    rank2 = c2 - sel2
    count1 = c1[-1:, :]
    count2 = c2[-1:, :]
    count = count1 + count2
    tiles = jnp.floor((count + (TILE - 1)) * (1.0 / TILE))
    lo = lax.broadcasted_iota(jnp.int32, (E, E), 0)
    hi = lax.broadcasted_iota(jnp.int32, (E, E), 1)
    cumt = jnp.dot(tiles, (lo <= hi).astype(jnp.float32),
                   preferred_element_type=jnp.float32)
    row_start = TILE * (cumt - tiles)

    p1 = jnp.sum(sel1 * (row_start + rank1), axis=1, keepdims=True)
    p2 = jnp.sum(sel2 * (row_start + count1 + rank2), axis=1, keepdims=True)
    pos_ref[...] = jnp.where(col == 0, p1.astype(jnp.int32),
                             jnp.where(col == 1, p2.astype(jnp.int32), 0))

    ti = lax.broadcasted_iota(jnp.int32, (NTR, E), 0).astype(jnp.float32)
    te = jnp.sum((ti >= cumt).astype(jnp.int32), axis=1, keepdims=True)
    te_ref[...] = jnp.broadcast_to(te, (NTR, E))


def _scatter_kernel(pos_hbm, src_hbm, pos_v, src_v):
    wid = lax.axis_index("s") * NC + lax.axis_index("c")

    @pl.when(wid == 0)
    def _():
        pltpu.sync_copy(pos_hbm, pos_v)
        iota16 = lax.iota(jnp.int32, 16)

        @pl.loop(0, ROWS, step=16)
        def _(i):
            src_v[pl.ds(i, 16)] = (iota16 + i) & (T - 1)

        @pl.loop(0, T, step=16)
        def _(i):
            tok = iota16 + i
            p1 = plsc.load_gather(pos_v, [tok * E])
            plsc.store_scatter(src_v, [p1], tok)
            p2 = plsc.load_gather(pos_v, [tok * E + 1])
            plsc.store_scatter(src_v, [p2], tok)

        pltpu.sync_copy(src_v, src_hbm)


def _gather_kernel(table_hbm, idx_hbm, out_hbm, idx_v, rows_v, sem):
    wid = lax.axis_index("s") * NC + lax.axis_index("c")
    per_w = ROWS // NW
    chunk = 64
    base = wid * per_w
    pltpu.sync_copy(idx_hbm.at[pl.ds(base, per_w)], idx_v)
    for k in range(per_w // chunk):
        pltpu.async_copy(table_hbm.at[idx_v.at[pl.ds(k * chunk, chunk)]],
                         rows_v, sem).wait()
        pltpu.sync_copy(rows_v, out_hbm.at[pl.ds(base + k * chunk, chunk)])


def _ffn_body(x_ref, wg_ref, wu_ref, wd_ref, o_ref):
    xb = x_ref[...].astype(jnp.bfloat16)
    g = jnp.dot(xb, wg_ref[0].T, preferred_element_type=jnp.float32)
    u = jnp.dot(xb, wu_ref[0].T, preferred_element_type=jnp.float32)
    hmid = (g * jax.nn.sigmoid(g)) * u
    o_ref[...] = jnp.dot(hmid.astype(jnp.bfloat16), wd_ref[0].T,
                         preferred_element_type=jnp.float32)


def _ffn_routed_kernel(pf_ref, x_ref, wg_ref, wu_ref, wd_ref, o_ref):
    @pl.when(pf_ref[pl.program_id(0)] < E)
    def _():
        _ffn_body(x_ref, wg_ref, wu_ref, wd_ref, o_ref)


def _combine_kernel(pw_ref, r1_ref, r2_ref, sh_ref, o_ref):
    col = lax.broadcasted_iota(jnp.int32, (SH_TILE, E), 1)
    pw = pw_ref[...]
    w1 = jnp.sum(jnp.where(col == 0, pw, 0.0), axis=1, keepdims=True)
    w2 = jnp.sum(jnp.where(col == 1, pw, 0.0), axis=1, keepdims=True)
    o_ref[...] = w1 * r1_ref[...] + w2 * r2_ref[...] + sh_ref[...]


@jax.jit
def kernel(x, Wg, We_gate, We_up, We_down, Ws_gate, Ws_up, Ws_down):
    xf = x.reshape(T, H)

    pw, pos, te_mat = pl.pallas_call(
        _router_kernel,
        out_shape=(
            jax.ShapeDtypeStruct((T, E), jnp.float32),
            jax.ShapeDtypeStruct((T, E), jnp.int32),
            jax.ShapeDtypeStruct((NTR, E), jnp.int32),
        ),
    )(xf, Wg)

    mesh = plsc.VectorSubcoreMesh(core_axis_name="c", subcore_axis_name="s")
    sc_params = pltpu.CompilerParams()
    if "needs_layout_passes" in pltpu.CompilerParams.__dataclass_fields__:
        sc_params = dataclasses.replace(sc_params, needs_layout_passes=False)

    src = pl.kernel(
        _scatter_kernel,
        out_type=jax.ShapeDtypeStruct((ROWS,), jnp.int32),
        mesh=mesh,
        scratch_types=[pltpu.VMEM((T * E,), jnp.int32),
                       pltpu.VMEM((ROWS,), jnp.int32)],
        compiler_params=sc_params,
    )(pos.reshape(-1))

    x_sorted = pl.kernel(
        _gather_kernel,
        out_type=jax.ShapeDtypeStruct((ROWS, H), jnp.float32),
        mesh=mesh,
        scratch_types=[pltpu.VMEM((ROWS // NW,), jnp.int32),
                       pltpu.VMEM((64, H), jnp.float32),
                       pltpu.SemaphoreType.DMA],
        compiler_params=sc_params,
    )(xf, src)

    te = te_mat[:, 0]
    pf = jnp.minimum(te, E).astype(jnp.int32)

    we_g = We_gate.astype(jnp.bfloat16)
    we_u = We_up.astype(jnp.bfloat16)
    we_d = We_down.astype(jnp.bfloat16)

    sh = pl.pallas_call(
        _ffn_body,
        grid=(T // SH_TILE,),
        in_specs=[
            pl.BlockSpec((SH_TILE, H), lambda i: (i, 0)),
            pl.BlockSpec((1, I, H), lambda i: (0, 0, 0)),
            pl.BlockSpec((1, I, H), lambda i: (0, 0, 0)),
            pl.BlockSpec((1, H, I), lambda i: (0, 0, 0)),
        ],
        out_specs=pl.BlockSpec((SH_TILE, H), lambda i: (i, 0)),
        out_shape=jax.ShapeDtypeStruct((T, H), jnp.float32),
    )(xf, Ws_gate.astype(jnp.bfloat16)[None],
      Ws_up.astype(jnp.bfloat16)[None], Ws_down.astype(jnp.bfloat16)[None])

    os_ = pl.pallas_call(
        _ffn_routed_kernel,
        grid_spec=pltpu.PrefetchScalarGridSpec(
            num_scalar_prefetch=1,
            grid=(NTR,),
            in_specs=[
                pl.BlockSpec((TILE, H), lambda i, pf: (i, 0)),
                pl.BlockSpec((1, I, H),
                             lambda i, pf: (jnp.minimum(pf[i], E - 1), 0, 0)),
                pl.BlockSpec((1, I, H),
                             lambda i, pf: (jnp.minimum(pf[i], E - 1), 0, 0)),
                pl.BlockSpec((1, H, I),
                             lambda i, pf: (jnp.minimum(pf[i], E - 1), 0, 0)),
            ],
            out_specs=pl.BlockSpec((TILE, H), lambda i, pf: (i, 0)),
        ),
        out_shape=jax.ShapeDtypeStruct((ROWS, H), jnp.float32),
        compiler_params=pltpu.CompilerParams(
            dimension_semantics=("arbitrary",),
        ),
    )(pf, x_sorted, we_g, we_u, we_d)

    pos1 = pos[:, 0]
    pos2 = pos[:, 1]

    def _pair_gather(os_hbm, i1_hbm, i2_hbm, r1_hbm, r2_hbm,
                     idx_v, rows_v, sem):
        wid = lax.axis_index("s") * NC + lax.axis_index("c")
        base = wid * (T // NW)
        pltpu.sync_copy(i1_hbm.at[pl.ds(base, T // NW)], idx_v)
        pltpu.async_copy(os_hbm.at[idx_v], rows_v, sem).wait()
        pltpu.sync_copy(rows_v, r1_hbm.at[pl.ds(base, T // NW)])
        pltpu.sync_copy(i2_hbm.at[pl.ds(base, T // NW)], idx_v)
        pltpu.async_copy(os_hbm.at[idx_v], rows_v, sem).wait()
        pltpu.sync_copy(rows_v, r2_hbm.at[pl.ds(base, T // NW)])

    r1, r2 = pl.kernel(
        _pair_gather,
        out_type=(jax.ShapeDtypeStruct((T, H), jnp.float32),
                  jax.ShapeDtypeStruct((T, H), jnp.float32)),
        mesh=mesh,
        scratch_types=[pltpu.VMEM((T // NW,), jnp.int32),
                       pltpu.VMEM((T // NW, H), jnp.float32),
                       pltpu.SemaphoreType.DMA],
        compiler_params=sc_params,
    )(os_, pos1, pos2)

    y = pl.pallas_call(
        _combine_kernel,
        grid=(T // SH_TILE,),
        in_specs=[
            pl.BlockSpec((SH_TILE, E), lambda i: (i, 0)),
            pl.BlockSpec((SH_TILE, H), lambda i: (i, 0)),
            pl.BlockSpec((SH_TILE, H), lambda i: (i, 0)),
            pl.BlockSpec((SH_TILE, H), lambda i: (i, 0)),
        ],
        out_specs=pl.BlockSpec((SH_TILE, H), lambda i: (i, 0)),
        out_shape=jax.ShapeDtypeStruct((T, H), jnp.float32),
    )(pw, r1, r2, sh)
    return y.reshape(B, S, H)


# speedup vs baseline: 2.9563x; 1.1980x over previous
import jax
import jax.numpy as jnp
from jax import lax
from jax.experimental import pallas as pl
from jax.experimental.pallas import tpu as pltpu

B, S, H = 1, 2048, 1024
E, K, I = 8, 2, 512
T = B * S
EP = 16
TT = 256
NT = T // TT


def _router_kernel(x_ref, wg_ref, w16_ref):
    logits = jnp.dot(x_ref[...], wg_ref[...].T,
                     preferred_element_type=jnp.float32)
    m1 = jnp.max(logits, axis=-1, keepdims=True)
    masked = jnp.where(logits == m1, -jnp.inf, logits)
    m2 = jnp.max(masked, axis=-1, keepdims=True)
    sel = logits >= m2
    e = jnp.where(sel, jnp.exp(logits - m1), 0.0)
    w = e / jnp.sum(e, axis=-1, keepdims=True)
    w16 = jnp.pad(w, ((0, 0), (0, EP - E)))
    col = lax.broadcasted_iota(jnp.int32, (T, EP), 1)
    w16_ref[...] = jnp.where(col == E, 1.0, w16)


def _moe_kernel(x_ref, w16_ref, wg_ref, wu_ref, wd_ref, o_ref):
    xb = x_ref[...]
    w16 = w16_ref[...]
    col = lax.broadcasted_iota(jnp.int32, (TT, EP), 1)
    acc = None
    for e in range(E + 1):
        g = jnp.dot(xb, wg_ref[e].T, preferred_element_type=jnp.float32)
        u = jnp.dot(xb, wu_ref[e].T, preferred_element_type=jnp.float32)
        hm = (g * jax.nn.sigmoid(g)) * u
        we = jnp.sum(jnp.where(col == e, w16, 0.0), axis=1, keepdims=True)
        hm = hm * we
        part = jnp.dot(hm.astype(jnp.bfloat16), wd_ref[e].T,
                       preferred_element_type=jnp.float32)
        acc = part if acc is None else acc + part
    o_ref[...] = acc


@jax.jit
def kernel(x, Wg, We_gate, We_up, We_down, Ws_gate, Ws_up, Ws_down):
    xf = x.reshape(T, H)
    w16 = pl.pallas_call(
        _router_kernel,
        out_shape=jax.ShapeDtypeStruct((T, EP), jnp.float32),
    )(xf, Wg)

    wcat_g = jnp.concatenate([We_gate, Ws_gate[None]], 0).astype(jnp.bfloat16)
    wcat_u = jnp.concatenate([We_up, Ws_up[None]], 0).astype(jnp.bfloat16)
    wcat_d = jnp.concatenate([We_down, Ws_down[None]], 0).astype(jnp.bfloat16)
    xbf = xf.astype(jnp.bfloat16)

    y = pl.pallas_call(
        _moe_kernel,
        grid=(NT,),
        in_specs=[
            pl.BlockSpec((TT, H), lambda t: (t, 0)),
            pl.BlockSpec((TT, EP), lambda t: (t, 0)),
            pl.BlockSpec((E + 1, I, H), lambda t: (0, 0, 0)),
            pl.BlockSpec((E + 1, I, H), lambda t: (0, 0, 0)),
            pl.BlockSpec((E + 1, H, I), lambda t: (0, 0, 0)),
        ],
        out_specs=pl.BlockSpec((TT, H), lambda t: (t, 0)),
        out_shape=jax.ShapeDtypeStruct((T, H), jnp.float32),
    )(xbf, w16, wcat_g, wcat_u, wcat_d)
    return y.reshape(B, S, H)
